# baseline (speedup 1.0000x reference)
import functools

import jax
import jax.numpy as jnp
from jax import lax
from jax.experimental import pallas as pl
from jax.experimental.pallas import tpu as pltpu
from jax.experimental.pallas import tpu_sc as plsc

B = 16384
K = 64
KP = 2 * K
DIM_PAIR = 50000
NOUT = 256
NC = 2
NS = 16
NW = NC * NS
BPW = B // NW
CH = 128
NCH = BPW // CH
LANES = 16


def _sc_gather_prod_kernel(pi0_hbm, pi1_hbm, pi2_hbm, par_hbm, t0_hbm, t1_hbm,
                           t2_hbm, out_hbm, i0_v, i1_v, i2_v, par_v,
                           r0_v, r1_v, r2_v, o_v, s0, s1, s2):
    wid = lax.axis_index("s") * NC + lax.axis_index("c")
    base = wid * BPW
    pltpu.sync_copy(pi0_hbm.at[pl.ds(base, BPW)], i0_v)
    pltpu.sync_copy(pi1_hbm.at[pl.ds(base, BPW)], i1_v)
    pltpu.sync_copy(pi2_hbm.at[pl.ds(base, BPW)], i2_v)
    pltpu.sync_copy(par_hbm.at[pl.ds(base, BPW)], par_v)

    for ch in range(NCH):
        sl = pl.ds(ch * CH, CH)
        c0 = pltpu.async_copy(t0_hbm.at[i0_v.at[sl]], r0_v, s0)
        c1 = pltpu.async_copy(t1_hbm.at[i1_v.at[sl]], r1_v, s1)
        c2 = pltpu.async_copy(t2_hbm.at[i2_v.at[sl]], r2_v, s2)
        c0.wait()
        c1.wait()
        c2.wait()

        def body(g, carry):
            p_vec = par_v[pl.ds(ch * CH + g * LANES, LANES)]
            for j in range(LANES):
                p = p_vec[j]
                p0 = p & 0xFF
                p1 = (p >> 8) & 0xFF
                p2 = (p >> 16) & 0xFF
                r = g * LANES + j
                for c in range(K // LANES):
                    o_v[r, pl.ds(c * LANES, LANES)] = (
                        r0_v[r, pl.ds(p0 + c * LANES, LANES)]
                        * r1_v[r, pl.ds(p1 + c * LANES, LANES)]
                        * r2_v[r, pl.ds(p2 + c * LANES, LANES)]
                    )
            return carry

        lax.fori_loop(0, CH // LANES, body, 0)
        pltpu.sync_copy(o_v, out_hbm.at[pl.ds(base + ch * CH, CH)])


@jax.jit
def _sc_gather_prod(pi0, pi1, pi2, par, t0, t1, t2):
    mesh = plsc.VectorSubcoreMesh(core_axis_name="c", subcore_axis_name="s")
    return pl.kernel(
        _sc_gather_prod_kernel,
        mesh=mesh,
        compiler_params=pltpu.CompilerParams(use_tc_tiling_on_sc=True),
        out_type=jax.ShapeDtypeStruct((B, K), jnp.float32),
        scratch_types=[
            pltpu.VMEM((BPW,), jnp.int32),
            pltpu.VMEM((BPW,), jnp.int32),
            pltpu.VMEM((BPW,), jnp.int32),
            pltpu.VMEM((BPW,), jnp.int32),
            pltpu.VMEM((CH, KP), jnp.float32),
            pltpu.VMEM((CH, KP), jnp.float32),
            pltpu.VMEM((CH, KP), jnp.float32),
            pltpu.VMEM((CH, K), jnp.float32),
            pltpu.SemaphoreType.DMA,
            pltpu.SemaphoreType.DMA,
            pltpu.SemaphoreType.DMA,
        ],
    )(pi0, pi1, pi2, par, t0, t1, t2)


BM = 2048


def _tc_proj_kernel(prod_ref, f3_ref, ls_ref, out_ref, ls_out_ref):
    out_ref[...] = lax.dot_general(
        prod_ref[...], f3_ref[...],
        dimension_numbers=(((1,), (1,)), ((), ())),
        preferred_element_type=jnp.float32,
    )
    ls_out_ref[...] = jnp.clip(ls_ref[...], -2.5, 0.0)


@jax.jit
def _tc_proj(prod, f3, log_sigma):
    return pl.pallas_call(
        _tc_proj_kernel,
        grid=(B // BM,),
        in_specs=[
            pl.BlockSpec((BM, K), lambda i: (i, 0)),
            pl.BlockSpec((NOUT, K), lambda i: (0, 0)),
            pl.BlockSpec((1, NOUT), lambda i: (0, 0)),
        ],
        out_specs=[
            pl.BlockSpec((BM, NOUT), lambda i: (i, 0)),
            pl.BlockSpec((1, NOUT), lambda i: (0, 0)),
        ],
        out_shape=[
            jax.ShapeDtypeStruct((B, NOUT), jnp.float32),
            jax.ShapeDtypeStruct((1, NOUT), jnp.float32),
        ],
    )(prod, f3, log_sigma)


def kernel(indices, f0, f1, f2, f3, log_sigma):
    idx = indices.astype(jnp.int32)
    pair = idx >> 1
    half = (idx & 1) * K
    par = half[:, 0] | (half[:, 1] << 8) | (half[:, 2] << 16)
    prod = _sc_gather_prod(
        pair[:, 0], pair[:, 1], pair[:, 2], par,
        f0.reshape(DIM_PAIR, KP), f1.reshape(DIM_PAIR, KP),
        f2.reshape(DIM_PAIR, KP),
    )
    res, ls = _tc_proj(prod, f3, log_sigma)
    return (res, ls)


# speedup vs baseline: 1.2459x; 1.2459x over previous
import functools

import jax
import jax.numpy as jnp
from jax import lax
from jax.experimental import pallas as pl
from jax.experimental.pallas import tpu as pltpu
from jax.experimental.pallas import tpu_sc as plsc

B = 16384
K = 64
NOUT = 256
NC = 2
NS = 16
NW = NC * NS
BPW = B // NW
G = 16
LANES = 16


def _sc_gather_prod_kernel(i0_hbm, i1_hbm, i2_hbm, t0_hbm, t1_hbm, t2_hbm,
                           out_hbm, i0_v, i1_v, i2_v, r1_v, r2_v, o_v,
                           s0, s1, s2):
    wid = lax.axis_index("s") * NC + lax.axis_index("c")
    base = wid * BPW
    pltpu.sync_copy(i0_hbm.at[pl.ds(base, BPW)], i0_v)
    pltpu.sync_copy(i1_hbm.at[pl.ds(base, BPW)], i1_v)
    pltpu.sync_copy(i2_hbm.at[pl.ds(base, BPW)], i2_v)

    def group(g, carry):
        v0 = i0_v[pl.ds(g * G, G)]
        v1 = i1_v[pl.ds(g * G, G)]
        v2 = i2_v[pl.ds(g * G, G)]
        cps = []
        for j in range(G):
            cps.append(pltpu.async_copy(t0_hbm.at[v0[j]], o_v.at[g * G + j], s0))
            cps.append(pltpu.async_copy(t1_hbm.at[v1[j]], r1_v.at[j], s1))
            cps.append(pltpu.async_copy(t2_hbm.at[v2[j]], r2_v.at[j], s2))
        for cp in cps:
            cp.wait()
        for j in range(G):
            r = g * G + j
            for c in range(K // LANES):
                sl = pl.ds(c * LANES, LANES)
                o_v[r, sl] = o_v[r, sl] * r1_v[j, sl] * r2_v[j, sl]
        return carry

    lax.fori_loop(0, BPW // G, group, 0)
    pltpu.sync_copy(o_v, out_hbm.at[pl.ds(base, BPW)])


@jax.jit
def _sc_gather_prod(i0, i1, i2, t0, t1, t2):
    mesh = plsc.VectorSubcoreMesh(core_axis_name="c", subcore_axis_name="s")
    return pl.kernel(
        _sc_gather_prod_kernel,
        mesh=mesh,
        compiler_params=pltpu.CompilerParams(use_tc_tiling_on_sc=True),
        out_type=jax.ShapeDtypeStruct((B, K), jnp.float32),
        scratch_types=[
            pltpu.VMEM((BPW,), jnp.int32),
            pltpu.VMEM((BPW,), jnp.int32),
            pltpu.VMEM((BPW,), jnp.int32),
            pltpu.VMEM((G, K), jnp.float32),
            pltpu.VMEM((G, K), jnp.float32),
            pltpu.VMEM((BPW, K), jnp.float32),
            pltpu.SemaphoreType.DMA,
            pltpu.SemaphoreType.DMA,
            pltpu.SemaphoreType.DMA,
        ],
    )(i0, i1, i2, t0, t1, t2)


BM = 2048


def _tc_proj_kernel(prod_ref, f3_ref, ls_ref, out_ref, ls_out_ref):
    out_ref[...] = lax.dot_general(
        prod_ref[...], f3_ref[...],
        dimension_numbers=(((1,), (1,)), ((), ())),
        preferred_element_type=jnp.float32,
    )
    ls_out_ref[...] = jnp.clip(ls_ref[...], -2.5, 0.0)


@jax.jit
def _tc_proj(prod, f3, log_sigma):
    return pl.pallas_call(
        _tc_proj_kernel,
        grid=(B // BM,),
        in_specs=[
            pl.BlockSpec((BM, K), lambda i: (i, 0)),
            pl.BlockSpec((NOUT, K), lambda i: (0, 0)),
            pl.BlockSpec((1, NOUT), lambda i: (0, 0)),
        ],
        out_specs=[
            pl.BlockSpec((BM, NOUT), lambda i: (i, 0)),
            pl.BlockSpec((1, NOUT), lambda i: (0, 0)),
        ],
        out_shape=[
            jax.ShapeDtypeStruct((B, NOUT), jnp.float32),
            jax.ShapeDtypeStruct((1, NOUT), jnp.float32),
        ],
    )(prod, f3, log_sigma)


def kernel(indices, f0, f1, f2, f3, log_sigma):
    idx = indices.astype(jnp.int32)
    prod = _sc_gather_prod(idx[:, 0], idx[:, 1], idx[:, 2], f0, f1, f2)
    res, ls = _tc_proj(prod, f3, log_sigma)
    return (res, ls)


# speedup vs baseline: 1.7106x; 1.3729x over previous
import functools

import jax
import jax.numpy as jnp
from jax import lax
from jax.experimental import pallas as pl
from jax.experimental.pallas import tpu as pltpu
from jax.experimental.pallas import tpu_sc as plsc

B = 16384
K = 64
DIM = 100000
NOUT = 256
NC = 2
NS = 16
NW = NC * NS
FPW = K // NW
ICH = 8192
LANES = 16


def _sc_gather_prod_kernel(i0_hbm, i1_hbm, i2_hbm, t0_hbm, t1_hbm, t2_hbm,
                           out_hbm, row_v, acc_v, idx_v, sdma):
    wid = lax.axis_index("s") * NC + lax.axis_index("c")

    for f in range(FPW):
        k = wid * FPW + f
        for t, t_hbm, i_hbm in ((0, t0_hbm, i0_hbm), (1, t1_hbm, i1_hbm),
                                (2, t2_hbm, i2_hbm)):
            pltpu.async_copy(t_hbm.at[k, pl.ds(0, DIM)], row_v, sdma).wait()
            for ci in range(B // ICH):
                pltpu.sync_copy(i_hbm.at[pl.ds(ci * ICH, ICH)], idx_v)

                def vloop(v, carry, _t=t, _ci=ci):
                    iv = idx_v[pl.ds(v * LANES, LANES)]
                    g = plsc.load_gather(row_v, [iv])
                    off = _ci * ICH + v * LANES
                    if _t == 0:
                        acc_v[pl.ds(off, LANES)] = g
                    else:
                        acc_v[pl.ds(off, LANES)] = acc_v[pl.ds(off, LANES)] * g
                    return carry

                lax.fori_loop(0, ICH // LANES, vloop, 0, unroll=4)
        pltpu.sync_copy(acc_v, out_hbm.at[k, pl.ds(0, B)])


@jax.jit
def _sc_gather_prod(i0, i1, i2, t0t, t1t, t2t):
    mesh = plsc.VectorSubcoreMesh(core_axis_name="c", subcore_axis_name="s")
    return pl.kernel(
        _sc_gather_prod_kernel,
        mesh=mesh,
        compiler_params=pltpu.CompilerParams(use_tc_tiling_on_sc=True,
                                             needs_layout_passes=False),
        out_type=jax.ShapeDtypeStruct((K, B), jnp.float32),
        scratch_types=[
            pltpu.VMEM((DIM,), jnp.float32),
            pltpu.VMEM((B,), jnp.float32),
            pltpu.VMEM((ICH,), jnp.int32),
            pltpu.SemaphoreType.DMA,
        ],
    )(i0, i1, i2, t0t, t1t, t2t)


BM = 2048


def _tc_proj_kernel(prodt_ref, f3_ref, ls_ref, out_ref, ls_out_ref):
    out_ref[...] = lax.dot_general(
        prodt_ref[...], f3_ref[...],
        dimension_numbers=(((0,), (1,)), ((), ())),
        preferred_element_type=jnp.float32,
    )
    ls_out_ref[...] = jnp.clip(ls_ref[...], -2.5, 0.0)


@jax.jit
def _tc_proj(prodt, f3, log_sigma):
    return pl.pallas_call(
        _tc_proj_kernel,
        grid=(B // BM,),
        in_specs=[
            pl.BlockSpec((K, BM), lambda i: (0, i)),
            pl.BlockSpec((NOUT, K), lambda i: (0, 0)),
            pl.BlockSpec((1, NOUT), lambda i: (0, 0)),
        ],
        out_specs=[
            pl.BlockSpec((BM, NOUT), lambda i: (i, 0)),
            pl.BlockSpec((1, NOUT), lambda i: (0, 0)),
        ],
        out_shape=[
            jax.ShapeDtypeStruct((B, NOUT), jnp.float32),
            jax.ShapeDtypeStruct((1, NOUT), jnp.float32),
        ],
    )(prodt, f3, log_sigma)


def kernel(indices, f0, f1, f2, f3, log_sigma):
    idx = indices.astype(jnp.int32)
    prodt = _sc_gather_prod(idx[:, 0], idx[:, 1], idx[:, 2],
                            f0.T, f1.T, f2.T)
    res, ls = _tc_proj(prodt, f3, log_sigma)
    return (res, ls)


# speedup vs baseline: 2.5966x; 1.5180x over previous
import functools

import jax
import jax.numpy as jnp
from jax import lax
from jax.experimental import pallas as pl
from jax.experimental.pallas import tpu as pltpu
from jax.experimental.pallas import tpu_sc as plsc

B = 16384
K = 64
DIM = 100000
NOUT = 256
NC = 2
NS = 16
NW = NC * NS
FPW = K // NW
ICH = 8192
LANES = 16


def _sc_gather_prod_kernel(i0_hbm, i1_hbm, i2_hbm, t0_hbm, t1_hbm, t2_hbm,
                           out_hbm, row_v, acc_v, idx_v, sdma):
    wid = lax.axis_index("s") * NC + lax.axis_index("c")

    for f in range(FPW):
        k = wid * FPW + f
        for t, t_hbm, i_hbm in ((0, t0_hbm, i0_hbm), (1, t1_hbm, i1_hbm),
                                (2, t2_hbm, i2_hbm)):
            pltpu.async_copy(t_hbm.at[k, pl.ds(0, DIM)], row_v, sdma).wait()
            for ci in range(B // ICH):
                pltpu.sync_copy(i_hbm.at[pl.ds(ci * ICH, ICH)], idx_v)

                @plsc.parallel_loop(0, ICH // LANES, unroll=8)
                def vloop(v, _t=t, _ci=ci):
                    iv = idx_v[pl.ds(v * LANES, LANES)]
                    g = plsc.load_gather(row_v, [iv])
                    off = _ci * ICH + v * LANES
                    if _t == 0:
                        acc_v[pl.ds(off, LANES)] = g
                    else:
                        acc_v[pl.ds(off, LANES)] = acc_v[pl.ds(off, LANES)] * g
        pltpu.sync_copy(acc_v, out_hbm.at[k, pl.ds(0, B)])


@jax.jit
def _sc_gather_prod(i0, i1, i2, t0t, t1t, t2t):
    mesh = plsc.VectorSubcoreMesh(core_axis_name="c", subcore_axis_name="s")
    return pl.kernel(
        _sc_gather_prod_kernel,
        mesh=mesh,
        compiler_params=pltpu.CompilerParams(use_tc_tiling_on_sc=True,
                                             needs_layout_passes=False),
        out_type=jax.ShapeDtypeStruct((K, B), jnp.float32),
        scratch_types=[
            pltpu.VMEM((DIM,), jnp.float32),
            pltpu.VMEM((B,), jnp.float32),
            pltpu.VMEM((ICH,), jnp.int32),
            pltpu.SemaphoreType.DMA,
        ],
    )(i0, i1, i2, t0t, t1t, t2t)


BM = 2048


def _tc_proj_kernel(prodt_ref, f3_ref, ls_ref, out_ref, ls_out_ref):
    out_ref[...] = lax.dot_general(
        prodt_ref[...], f3_ref[...],
        dimension_numbers=(((0,), (1,)), ((), ())),
        preferred_element_type=jnp.float32,
    )
    ls_out_ref[...] = jnp.clip(ls_ref[...], -2.5, 0.0)


@jax.jit
def _tc_proj(prodt, f3, log_sigma):
    return pl.pallas_call(
        _tc_proj_kernel,
        grid=(B // BM,),
        in_specs=[
            pl.BlockSpec((K, BM), lambda i: (0, i)),
            pl.BlockSpec((NOUT, K), lambda i: (0, 0)),
            pl.BlockSpec((1, NOUT), lambda i: (0, 0)),
        ],
        out_specs=[
            pl.BlockSpec((BM, NOUT), lambda i: (i, 0)),
            pl.BlockSpec((1, NOUT), lambda i: (0, 0)),
        ],
        out_shape=[
            jax.ShapeDtypeStruct((B, NOUT), jnp.float32),
            jax.ShapeDtypeStruct((1, NOUT), jnp.float32),
        ],
    )(prodt, f3, log_sigma)


def kernel(indices, f0, f1, f2, f3, log_sigma):
    idx = indices.astype(jnp.int32)
    prodt = _sc_gather_prod(idx[:, 0], idx[:, 1], idx[:, 2],
                            f0.T, f1.T, f2.T)
    res, ls = _tc_proj(prodt, f3, log_sigma)
    return (res, ls)
